# baseline (speedup 1.0000x reference)
import jax
import jax.numpy as jnp
from jax.experimental import pallas as pl
from jax.experimental.pallas import tpu as pltpu

NSEG = 8
FOLD = 24


def _body(x_ref, o_ref):
    xr = x_ref[0]
    o_ref[0, :, 0:1, :] = xr[:, 0:1, :]
    o_ref[0, :, FOLD + 1:-FOLD, :] = xr[:, FOLD + 1:-FOLD, :]
    o_ref[0, :, 1:FOLD + 1, :] = jnp.concatenate(
        [xr[NSEG - 1:, 1:FOLD + 1, :], xr[:NSEG - 1, 1:FOLD + 1, :]], axis=0)
    o_ref[0, :, -FOLD:, :] = jnp.concatenate(
        [xr[1:, -FOLD:, :], xr[:1, -FOLD:, :]], axis=0)


def kernel(x):
    nt, l, c = x.shape
    nb = nt // NSEG
    xr = x.reshape(nb, NSEG, l, c)
    out = pl.pallas_call(
        _body,
        grid=(nb,),
        in_specs=[pl.BlockSpec((1, NSEG, l, c), lambda i: (i, 0, 0, 0))],
        out_specs=pl.BlockSpec((1, NSEG, l, c), lambda i: (i, 0, 0, 0)),
        out_shape=jax.ShapeDtypeStruct((nb, NSEG, l, c), x.dtype),
    )(xr)
    return out.reshape(nt, l, c)
